# baseline (speedup 1.0000x reference)
import functools
import numpy as np
import jax
from jax import lax
import jax.numpy as jnp
from jax.experimental import pallas as pl
from jax.experimental.pallas import tpu as pltpu
from jax.experimental.pallas import tpu_sc as plsc

_Nr = 1000
_Ng = 32
_C = 200
_B = 8
_T = 4032

_FG_IOU = 0.7
_BG_IOU = 0.01

_NW = 32
_RPW = (_B * _Nr) // _NW

_sc_mesh = plsc.VectorSubcoreMesh(core_axis_name="c", subcore_axis_name="s")


_RPAD = 256
_WCH = _RPW * _C


@functools.partial(
    pl.kernel,
    mesh=_sc_mesh,
    compiler_params=pltpu.CompilerParams(needs_layout_passes=False),
    out_type=[
        jax.ShapeDtypeStruct((_NW * _RPAD,), jnp.float32),
        jax.ShapeDtypeStruct((_NW * _RPAD,), jnp.float32),
        jax.ShapeDtypeStruct((_NW * _RPAD,), jnp.float32),
        jax.ShapeDtypeStruct((_NW * _RPAD * _Ng,), jnp.float32),
    ],
    scratch_types=[
        pltpu.VMEM((_RPAD * _C,), jnp.float32),
        pltpu.VMEM((_Ng,), jnp.int32),
        pltpu.VMEM((_RPAD,), jnp.float32),
        pltpu.VMEM((_RPAD,), jnp.float32),
        pltpu.VMEM((_RPAD,), jnp.float32),
        pltpu.VMEM((_RPAD * _Ng,), jnp.float32),
    ],
)
def _sc_stats(cls_hbm, lab_hbm, rmax_hbm, esum_hbm, col0_hbm, pick_hbm,
              rows_v, lab_v, rmax_v, esum_v, col0_v, pick_v):
    wid = lax.axis_index("s") * 2 + lax.axis_index("c")
    pltpu.sync_copy(cls_hbm.at[pl.ds(wid * _WCH, _WCH)],
                    rows_v.at[pl.ds(0, _WCH)])
    pltpu.sync_copy(lab_hbm.at[pl.ds(wid * _Ng, _Ng)], lab_v)
    la = lab_v[pl.ds(0, 16)]
    lb = lab_v[pl.ds(16, 16)]
    lanes = lax.iota(jnp.int32, 16)
    tmask = lanes >= 8

    def grp_body(g, carry):
        base = g * 16
        mvec = jnp.zeros((16,), jnp.float32)
        svec = jnp.zeros((16,), jnp.float32)
        cvec = jnp.zeros((16,), jnp.float32)
        for l in range(16):
            r = base + l
            off = r * _C
            v0 = rows_v[pl.ds(off, 16)]
            vs = [v0]
            m = v0
            for k in range(1, _C // 16):
                vk = rows_v[pl.ds(off + 16 * k, 16)]
                vs.append(vk)
                m = jnp.maximum(m, vk)
            vt = rows_v[pl.ds(off + _C - 16, 16)]
            m = jnp.maximum(m, vt)
            for hop in (8, 4, 2, 1):
                perm = jnp.bitwise_xor(lanes, hop)
                m = jnp.maximum(m, m.at[perm].get(mode="promise_in_bounds"))
            s = jnp.exp(vs[0] - m)
            for vk in vs[1:]:
                s = s + jnp.exp(vk - m)
            s = s + jnp.where(tmask, jnp.exp(vt - m), 0.0)
            for hop in (8, 4, 2, 1):
                perm = jnp.bitwise_xor(lanes, hop)
                s = s + s.at[perm].get(mode="promise_in_bounds")
            isl = lanes == l
            mvec = jnp.where(isl, m, mvec)
            svec = jnp.where(isl, s, svec)
            cvec = jnp.where(isl, v0[0], cvec)
            osplat = jnp.full((16,), off, jnp.int32)
            pick_v[pl.ds(r * _Ng, 16)] = plsc.load_gather(rows_v, [osplat + la])
            pick_v[pl.ds(r * _Ng + 16, 16)] = plsc.load_gather(
                rows_v, [osplat + lb])
        rmax_v[pl.ds(base, 16)] = mvec
        esum_v[pl.ds(base, 16)] = svec
        col0_v[pl.ds(base, 16)] = cvec
        return carry

    lax.fori_loop(0, _RPAD // 16, grp_body, 0)
    pltpu.sync_copy(rmax_v, rmax_hbm.at[pl.ds(wid * _RPAD, _RPAD)])
    pltpu.sync_copy(esum_v, esum_hbm.at[pl.ds(wid * _RPAD, _RPAD)])
    pltpu.sync_copy(col0_v, col0_hbm.at[pl.ds(wid * _RPAD, _RPAD)])
    pltpu.sync_copy(pick_v, pick_hbm.at[pl.ds(wid * _RPAD * _Ng, _RPAD * _Ng)])


def _body(rr3_ref, segc_ref, tri32_ref, rme_ref, pick_ref, pts_ref,
          out_ref, acc_ref):
    j = pl.program_id(0)

    @pl.when(j == 0)
    def _init():
        x = pts_ref[0:8].astype(jnp.float32)
        g = pts_ref[8:16].astype(jnp.float32)
        m = pts_ref[16:24].astype(jnp.float32)
        t = (g > 0.5).astype(jnp.float32)
        ax = jnp.abs(x)
        l1p = jnp.log1p(jnp.exp(-ax))
        ls_pos = jnp.minimum(x, 0.0) - l1p
        ls_neg = jnp.minimum(-x, 0.0) - l1p
        ce_f = -(t * ls_pos + (1.0 - t) * ls_neg)
        p = 1.0 / (1.0 + jnp.exp(-x))
        p_t = p * t + (1.0 - p) * (1.0 - t)
        q = 1.0 - p_t
        alpha_t = 0.25 * t + 0.75 * (1.0 - t)
        fl = alpha_t * ce_f * q * q
        acc_ref[2] = jnp.sum(fl * m)
        posm = t * m
        acc_ref[4] = jnp.sum(posm)

        lp = pts_ref[24:32].astype(jnp.float32)
        rp = pts_ref[32:40].astype(jnp.float32)
        lg = pts_ref[40:48].astype(jnp.float32)
        rg = pts_ref[48:56].astype(jnp.float32)
        intsctk = jnp.minimum(rp, rg) + jnp.minimum(lp, lg)
        unionk = (lp + rp) + (lg + rg) - intsctk
        iouk = intsctk / jnp.maximum(unionk, 1e-8)
        len_c = jnp.maximum(lp, lg) + jnp.maximum(rp, rg)
        rho = 0.5 * (rp - lp - rg + lg)
        rr_ = rho / jnp.maximum(len_c, 1e-8)
        dl = 1.0 - iouk + rr_ * rr_
        acc_ref[3] = jnp.sum(dl * posm)

        acc_ref[0] = 0.0
        acc_ref[1] = 0.0

    rl = rr3_ref[pl.ds(j, 1)]
    rr = rr3_ref[pl.ds(j + 8, 1)]
    sc = rr3_ref[pl.ds(j + 16, 1)]
    segc = segc_ref[0]
    gl = segc[:, 0:1]
    gr = segc[:, 1:2]
    min_l = jnp.minimum(gl, rl)
    max_l = jnp.maximum(gl, rl)
    min_r = jnp.minimum(gr, rr)
    max_r = jnp.maximum(gr, rr)
    mat = (min_r - max_l) / (max_r - min_l)
    ious = jnp.max(mat, axis=0, keepdims=True)
    ismax = mat >= ious
    ismax_bf = ismax.astype(jnp.bfloat16)
    cnt = jnp.dot(tri32_ref[...], ismax_bf,
                  preferred_element_type=jnp.float32)
    posf = (ious > _FG_IOU).astype(jnp.float32)
    npos = jnp.sum(posf)
    G = jnp.where(ismax & (cnt < 1.5), posf, 0.0)
    bgf = jnp.where((ious < _BG_IOU) & (sc > 0.0), 1.0, 0.0)
    cum = bgf
    for k in (1, 2, 4, 8, 16, 32, 64, 128, 256, 512):
        cum = cum + jnp.concatenate(
            [jnp.zeros((1, k), jnp.float32), cum[:, :-k]], axis=1)
    bg_sel = bgf * (cum < npos + 0.5).astype(jnp.float32)
    sel = jnp.maximum(posf, bg_sel)

    rmaxr = rme_ref[pl.ds(j, 1)]
    esumr = rme_ref[pl.ds(j + 8, 1)]
    col0r = rme_ref[pl.ds(j + 16, 1)]
    lse = rmaxr + jnp.log(esumr)
    s_lse = jnp.sum(sel * lse)
    s_bg = jnp.sum(bg_sel * col0r)
    s_pos = jnp.sum(G * pick_ref[0])
    acc_ref[0] = acc_ref[0] + (s_lse - s_bg - s_pos)
    acc_ref[1] = acc_ref[1] + jnp.sum(sel)

    @pl.when(j == _B - 1)
    def _fin():
        norm = 90.0 + 0.1 * jnp.maximum(acc_ref[4], 1.0)
        out_ref[0, 0] = (acc_ref[2] + acc_ref[3]) / norm + acc_ref[0] / acc_ref[1]


_TRI32 = np.tri(_Ng, dtype=np.float32).astype(jnp.bfloat16)


def kernel(fpn_masks, out_cls_logits, out_offsets, out_rois, out_scores,
           out_roimask, cls_log, gt_cls, gt_offsets, gt_segments,
           segments_label, segments_mask):
    f32 = jnp.float32
    tri32 = jnp.asarray(_TRI32)

    cls1 = cls_log.reshape(_B * _Nr * _C)
    lab_rep = jnp.repeat(segments_label.astype(jnp.int32), _NW // _B,
                         axis=0).reshape(_NW * _Ng)
    rmax, esum, col0, pick = _sc_stats(cls1, lab_rep)
    stats = jnp.stack([rmax, esum, col0], axis=0).reshape(
        3, _NW, _RPAD)[:, :, :_RPW]
    rme = stats.reshape(3 * _B, _Nr)
    pick_t = pick.reshape(_NW, _RPAD, _Ng)[:, :_RPW].reshape(
        _B, _NW // _B, _RPW, _Ng).transpose(
        0, 3, 1, 2).reshape(_B, _Ng, _Nr)

    rr3 = jnp.concatenate(
        [out_rois[:, :, 1], out_rois[:, :, 2], out_scores], axis=0)
    segc = jnp.stack(
        [gt_segments[:, :, 0], gt_segments[:, :, 1],
         segments_label.astype(f32)], axis=2)
    pts = jnp.concatenate(
        [out_cls_logits, gt_cls.astype(f32), fpn_masks.astype(f32),
         out_offsets[:, :, 0], out_offsets[:, :, 1],
         gt_offsets[:, :, 0], gt_offsets[:, :, 1]],
        axis=0).astype(jnp.bfloat16)

    out = pl.pallas_call(
        _body,
        grid=(_B,),
        in_specs=[
            pl.BlockSpec((3 * _B, _Nr), lambda j: (0, 0)),
            pl.BlockSpec((1, _Ng, 3), lambda j: (j, 0, 0)),
            pl.BlockSpec((_Ng, _Ng), lambda j: (0, 0)),
            pl.BlockSpec((3 * _B, _Nr), lambda j: (0, 0)),
            pl.BlockSpec((1, _Ng, _Nr), lambda j: (j, 0, 0)),
            pl.BlockSpec((7 * _B, _T), lambda j: (0, 0)),
        ],
        out_specs=pl.BlockSpec((1, 1), lambda j: (0, 0), memory_space=pltpu.SMEM),
        out_shape=jax.ShapeDtypeStruct((1, 1), f32),
        scratch_shapes=[pltpu.SMEM((8,), f32)],
    )(rr3, segc, tri32, rme, pick_t, pts)
    return out[0, 0]


# speedup vs baseline: 1.0234x; 1.0234x over previous
import functools
import numpy as np
import jax
from jax import lax
import jax.numpy as jnp
from jax.experimental import pallas as pl
from jax.experimental.pallas import tpu as pltpu
from jax.experimental.pallas import tpu_sc as plsc

_Nr = 1000
_Ng = 32
_C = 200
_B = 8
_T = 4032

_FG_IOU = 0.7
_BG_IOU = 0.01

_NW = 32
_RPW = (_B * _Nr) // _NW

_sc_mesh = plsc.VectorSubcoreMesh(core_axis_name="c", subcore_axis_name="s")


_RPAD = 256
_WCH = _RPW * _C


@functools.partial(
    pl.kernel,
    mesh=_sc_mesh,
    compiler_params=pltpu.CompilerParams(needs_layout_passes=False),
    out_type=[
        jax.ShapeDtypeStruct((_NW * _RPAD,), jnp.float32),
        jax.ShapeDtypeStruct((_NW * _RPAD,), jnp.float32),
        jax.ShapeDtypeStruct((_NW * _RPAD,), jnp.float32),
        jax.ShapeDtypeStruct((_NW * _RPAD * _Ng,), jnp.float32),
    ],
    scratch_types=[
        pltpu.VMEM((_RPAD * _C,), jnp.float32),
        pltpu.VMEM((_Ng,), jnp.int32),
        pltpu.VMEM((_RPAD,), jnp.float32),
        pltpu.VMEM((_RPAD,), jnp.float32),
        pltpu.VMEM((_RPAD,), jnp.float32),
        pltpu.VMEM((_RPAD * _Ng,), jnp.float32),
    ],
)
def _sc_stats(cls_hbm, lab_hbm, rmax_hbm, esum_hbm, col0_hbm, pick_hbm,
              rows_v, lab_v, rmax_v, esum_v, col0_v, pick_v):
    wid = lax.axis_index("s") * 2 + lax.axis_index("c")
    pltpu.sync_copy(cls_hbm.at[pl.ds(wid * _WCH, _WCH)],
                    rows_v.at[pl.ds(0, _WCH)])
    pltpu.sync_copy(lab_hbm.at[pl.ds((wid // 4) * _Ng, _Ng)], lab_v)
    la = lab_v[pl.ds(0, 16)]
    lb = lab_v[pl.ds(16, 16)]
    lanes = lax.iota(jnp.int32, 16)
    tmask = lanes >= 8

    def grp_body(g, carry):
        base = g * 16
        mvec = jnp.zeros((16,), jnp.float32)
        svec = jnp.zeros((16,), jnp.float32)
        cvec = jnp.zeros((16,), jnp.float32)
        for l in range(16):
            r = base + l
            off = r * _C
            v0 = rows_v[pl.ds(off, 16)]
            vs = [v0]
            m = v0
            for k in range(1, _C // 16):
                vk = rows_v[pl.ds(off + 16 * k, 16)]
                vs.append(vk)
                m = jnp.maximum(m, vk)
            vt = rows_v[pl.ds(off + _C - 16, 16)]
            m = jnp.maximum(m, vt)
            for hop in (8, 4, 2, 1):
                perm = jnp.bitwise_xor(lanes, hop)
                m = jnp.maximum(m, m.at[perm].get(mode="promise_in_bounds"))
            s = jnp.exp(vs[0] - m)
            for vk in vs[1:]:
                s = s + jnp.exp(vk - m)
            s = s + jnp.where(tmask, jnp.exp(vt - m), 0.0)
            for hop in (8, 4, 2, 1):
                perm = jnp.bitwise_xor(lanes, hop)
                s = s + s.at[perm].get(mode="promise_in_bounds")
            isl = lanes == l
            mvec = jnp.where(isl, m, mvec)
            svec = jnp.where(isl, s, svec)
            cvec = jnp.where(isl, v0[0], cvec)
        rmax_v[pl.ds(base, 16)] = mvec
        esum_v[pl.ds(base, 16)] = svec
        col0_v[pl.ds(base, 16)] = cvec
        rowoff = (jnp.full((16,), base, jnp.int32) + lanes) * _C
        for j in range(_Ng):
            labj = la[j] if j < 16 else lb[j - 16]
            pj = plsc.load_gather(rows_v, [rowoff + labj])
            pick_v[pl.ds(j * _RPAD + base, 16)] = pj
        return carry

    lax.fori_loop(0, _RPAD // 16, grp_body, 0)
    pltpu.sync_copy(rmax_v, rmax_hbm.at[pl.ds(wid * _RPAD, _RPAD)])
    pltpu.sync_copy(esum_v, esum_hbm.at[pl.ds(wid * _RPAD, _RPAD)])
    pltpu.sync_copy(col0_v, col0_hbm.at[pl.ds(wid * _RPAD, _RPAD)])
    pltpu.sync_copy(pick_v, pick_hbm.at[pl.ds(wid * _RPAD * _Ng, _RPAD * _Ng)])


def _body(rr3_ref, segc_ref, tri32_ref, rmax_ref, esum_ref, col0_ref,
          pick_ref, pts_ref, out_ref, acc_ref):
    j = pl.program_id(0)

    @pl.when(j == 0)
    def _init():
        x = pts_ref[0:8].astype(jnp.float32)
        g = pts_ref[8:16].astype(jnp.float32)
        m = pts_ref[16:24].astype(jnp.float32)
        t = (g > 0.5).astype(jnp.float32)
        ax = jnp.abs(x)
        l1p = jnp.log1p(jnp.exp(-ax))
        ls_pos = jnp.minimum(x, 0.0) - l1p
        ls_neg = jnp.minimum(-x, 0.0) - l1p
        ce_f = -(t * ls_pos + (1.0 - t) * ls_neg)
        p = 1.0 / (1.0 + jnp.exp(-x))
        p_t = p * t + (1.0 - p) * (1.0 - t)
        q = 1.0 - p_t
        alpha_t = 0.25 * t + 0.75 * (1.0 - t)
        fl = alpha_t * ce_f * q * q
        acc_ref[2] = jnp.sum(fl * m)
        posm = t * m
        acc_ref[4] = jnp.sum(posm)

        lp = pts_ref[24:32].astype(jnp.float32)
        rp = pts_ref[32:40].astype(jnp.float32)
        lg = pts_ref[40:48].astype(jnp.float32)
        rg = pts_ref[48:56].astype(jnp.float32)
        intsctk = jnp.minimum(rp, rg) + jnp.minimum(lp, lg)
        unionk = (lp + rp) + (lg + rg) - intsctk
        iouk = intsctk / jnp.maximum(unionk, 1e-8)
        len_c = jnp.maximum(lp, lg) + jnp.maximum(rp, rg)
        rho = 0.5 * (rp - lp - rg + lg)
        rr_ = rho / jnp.maximum(len_c, 1e-8)
        dl = 1.0 - iouk + rr_ * rr_
        acc_ref[3] = jnp.sum(dl * posm)

        acc_ref[0] = 0.0
        acc_ref[1] = 0.0

    rl = rr3_ref[pl.ds(j, 1)]
    rr = rr3_ref[pl.ds(j + 8, 1)]
    sc = rr3_ref[pl.ds(j + 16, 1)]
    segc = segc_ref[0]
    gl = segc[:, 0:1]
    gr = segc[:, 1:2]
    min_l = jnp.minimum(gl, rl)
    max_l = jnp.maximum(gl, rl)
    min_r = jnp.minimum(gr, rr)
    max_r = jnp.maximum(gr, rr)
    mat = (min_r - max_l) / (max_r - min_l)
    ious = jnp.max(mat, axis=0, keepdims=True)
    ismax = mat >= ious
    ismax_bf = ismax.astype(jnp.bfloat16)
    cnt = jnp.dot(tri32_ref[...], ismax_bf,
                  preferred_element_type=jnp.float32)
    posf = (ious > _FG_IOU).astype(jnp.float32)
    npos = jnp.sum(posf)
    G = jnp.where(ismax & (cnt < 1.5), posf, 0.0)
    bgf = jnp.where((ious < _BG_IOU) & (sc > 0.0), 1.0, 0.0)
    cum = bgf
    for k in (1, 2, 4, 8, 16, 32, 64, 128, 256, 512):
        cum = cum + jnp.concatenate(
            [jnp.zeros((1, k), jnp.float32), cum[:, :-k]], axis=1)
    bg_sel = bgf * (cum < npos + 0.5).astype(jnp.float32)
    sel = jnp.maximum(posf, bg_sel)

    ce_acc = jnp.float32(0.0)
    for q in range(4):
        mq = rmax_ref[0][:, q * _RPAD:q * _RPAD + 250]
        eq = esum_ref[0][:, q * _RPAD:q * _RPAD + 250]
        cq = col0_ref[0][:, q * _RPAD:q * _RPAD + 250]
        lseq = mq + jnp.log(eq)
        sel_q = sel[:, q * 250:(q + 1) * 250]
        bg_q = bg_sel[:, q * 250:(q + 1) * 250]
        G_q = G[:, q * 250:(q + 1) * 250]
        pq = pick_ref[0, q][:, 0:250]
        ce_acc = ce_acc + (jnp.sum(sel_q * lseq) - jnp.sum(bg_q * cq)
                           - jnp.sum(G_q * pq))
    acc_ref[0] = acc_ref[0] + ce_acc
    acc_ref[1] = acc_ref[1] + jnp.sum(sel)

    @pl.when(j == _B - 1)
    def _fin():
        norm = 90.0 + 0.1 * jnp.maximum(acc_ref[4], 1.0)
        out_ref[0, 0] = (acc_ref[2] + acc_ref[3]) / norm + acc_ref[0] / acc_ref[1]


_TRI32 = np.tri(_Ng, dtype=np.float32).astype(jnp.bfloat16)


def kernel(fpn_masks, out_cls_logits, out_offsets, out_rois, out_scores,
           out_roimask, cls_log, gt_cls, gt_offsets, gt_segments,
           segments_label, segments_mask):
    f32 = jnp.float32
    tri32 = jnp.asarray(_TRI32)

    cls1 = cls_log.reshape(_B * _Nr * _C)
    lab1 = segments_label.astype(jnp.int32).reshape(_B * _Ng)
    rmax, esum, col0, pick = _sc_stats(cls1, lab1)
    rmax8 = rmax.reshape(_B, 1, 4 * _RPAD)
    esum8 = esum.reshape(_B, 1, 4 * _RPAD)
    col08 = col0.reshape(_B, 1, 4 * _RPAD)
    pick4 = pick.reshape(_B, 4, _Ng, _RPAD)

    rr3 = jnp.concatenate(
        [out_rois[:, :, 1], out_rois[:, :, 2], out_scores], axis=0)
    segc = jnp.stack(
        [gt_segments[:, :, 0], gt_segments[:, :, 1],
         segments_label.astype(f32)], axis=2)
    pts = jnp.concatenate(
        [out_cls_logits, gt_cls.astype(f32), fpn_masks.astype(f32),
         out_offsets[:, :, 0], out_offsets[:, :, 1],
         gt_offsets[:, :, 0], gt_offsets[:, :, 1]],
        axis=0).astype(jnp.bfloat16)

    out = pl.pallas_call(
        _body,
        grid=(_B,),
        in_specs=[
            pl.BlockSpec((3 * _B, _Nr), lambda j: (0, 0)),
            pl.BlockSpec((1, _Ng, 3), lambda j: (j, 0, 0)),
            pl.BlockSpec((_Ng, _Ng), lambda j: (0, 0)),
            pl.BlockSpec((1, 1, 4 * _RPAD), lambda j: (j, 0, 0)),
            pl.BlockSpec((1, 1, 4 * _RPAD), lambda j: (j, 0, 0)),
            pl.BlockSpec((1, 1, 4 * _RPAD), lambda j: (j, 0, 0)),
            pl.BlockSpec((1, 4, _Ng, _RPAD), lambda j: (j, 0, 0, 0)),
            pl.BlockSpec((7 * _B, _T), lambda j: (0, 0)),
        ],
        out_specs=pl.BlockSpec((1, 1), lambda j: (0, 0), memory_space=pltpu.SMEM),
        out_shape=jax.ShapeDtypeStruct((1, 1), f32),
        scratch_shapes=[pltpu.SMEM((8,), f32)],
    )(rr3, segc, tri32, rmax8, esum8, col08, pick4, pts)
    return out[0, 0]


# speedup vs baseline: 1.8510x; 1.8087x over previous
import functools
import numpy as np
import jax
from jax import lax
import jax.numpy as jnp
from jax.experimental import pallas as pl
from jax.experimental.pallas import tpu as pltpu
from jax.experimental.pallas import tpu_sc as plsc

_Nr = 1000
_Ng = 32
_C = 200
_B = 8
_T = 4032

_FG_IOU = 0.7
_BG_IOU = 0.01

_NW = 32
_RPW = (_B * _Nr) // _NW

_sc_mesh = plsc.VectorSubcoreMesh(core_axis_name="c", subcore_axis_name="s")


_RPAD = 256
_QLEN = (248, 248, 248, 256)


@functools.partial(
    pl.kernel,
    mesh=_sc_mesh,
    compiler_params=pltpu.CompilerParams(needs_layout_passes=False),
    out_type=[
        jax.ShapeDtypeStruct((_B, 1, 4 * _RPAD), jnp.float32),
        jax.ShapeDtypeStruct((_B, 1, 4 * _RPAD), jnp.float32),
        jax.ShapeDtypeStruct((_B, 1, 4 * _RPAD), jnp.float32),
        jax.ShapeDtypeStruct((_B, 4, _Ng, _RPAD), jnp.float32),
    ],
    scratch_types=[
        pltpu.VMEM((_RPAD, _C), jnp.float32),
        pltpu.VMEM((_B, _Ng), jnp.int32),
        pltpu.VMEM((_RPAD,), jnp.float32),
        pltpu.VMEM((_RPAD,), jnp.float32),
        pltpu.VMEM((_RPAD,), jnp.float32),
        pltpu.VMEM((_Ng, _RPAD), jnp.float32),
    ],
)
def _sc_stats(cls_hbm, lab_hbm, rmax_hbm, esum_hbm, col0_hbm, pick_hbm,
              rows_v, lab_v, rmax_v, esum_v, col0_v, pick_v):
    wid = lax.axis_index("s") * 2 + lax.axis_index("c")
    b = wid // 4
    q = wid % 4
    st = q * 248
    pltpu.sync_copy(cls_hbm.at[b, pl.ds(st, _RPAD)], rows_v)
    pltpu.sync_copy(lab_hbm, lab_v)
    la = lab_v[b, pl.ds(0, 16)]
    lb = lab_v[b, pl.ds(16, 16)]
    lanes = lax.iota(jnp.int32, 16)
    tmask = lanes >= 8

    def grp_body(g, carry):
        base = g * 16
        mvec = jnp.zeros((16,), jnp.float32)
        svec = jnp.zeros((16,), jnp.float32)
        cvec = jnp.zeros((16,), jnp.float32)
        for l in range(16):
            r = base + l
            v0 = rows_v[r, pl.ds(0, 16)]
            vs = [v0]
            m = v0
            for k in range(1, _C // 16):
                vk = rows_v[r, pl.ds(16 * k, 16)]
                vs.append(vk)
                m = jnp.maximum(m, vk)
            vt = rows_v[r, pl.ds(_C - 16, 16)]
            m = jnp.maximum(m, vt)
            for hop in (8, 4, 2, 1):
                perm = jnp.bitwise_xor(lanes, hop)
                m = jnp.maximum(m, m.at[perm].get(mode="promise_in_bounds"))
            s = jnp.exp(vs[0] - m)
            for vk in vs[1:]:
                s = s + jnp.exp(vk - m)
            s = s + jnp.where(tmask, jnp.exp(vt - m), 0.0)
            for hop in (8, 4, 2, 1):
                perm = jnp.bitwise_xor(lanes, hop)
                s = s + s.at[perm].get(mode="promise_in_bounds")
            isl = lanes == l
            mvec = jnp.where(isl, m, mvec)
            svec = jnp.where(isl, s, svec)
            cvec = jnp.where(isl, v0[0], cvec)
        rmax_v[pl.ds(base, 16)] = mvec
        esum_v[pl.ds(base, 16)] = svec
        col0_v[pl.ds(base, 16)] = cvec
        rowvec = jnp.full((16,), base, jnp.int32) + lanes
        for j in range(_Ng):
            labj = la[j] if j < 16 else lb[j - 16]
            labv = jnp.full((16,), labj, jnp.int32)
            pick_v[j, pl.ds(base, 16)] = plsc.load_gather(
                rows_v, [rowvec, labv])
        return carry

    lax.fori_loop(0, _RPAD // 16, grp_body, 0)
    pltpu.sync_copy(rmax_v, rmax_hbm.at[b, 0, pl.ds(q * _RPAD, _RPAD)])
    pltpu.sync_copy(esum_v, esum_hbm.at[b, 0, pl.ds(q * _RPAD, _RPAD)])
    pltpu.sync_copy(col0_v, col0_hbm.at[b, 0, pl.ds(q * _RPAD, _RPAD)])
    pltpu.sync_copy(pick_v, pick_hbm.at[b, q])


def _body(rr3_ref, segc_ref, tri32_ref, rmax_ref, esum_ref, col0_ref,
          pick_ref, pts_ref, out_ref, acc_ref):
    j = pl.program_id(0)

    @pl.when(j == 0)
    def _init():
        x = pts_ref[0:8].astype(jnp.float32)
        g = pts_ref[8:16].astype(jnp.float32)
        m = pts_ref[16:24].astype(jnp.float32)
        t = (g > 0.5).astype(jnp.float32)
        ax = jnp.abs(x)
        l1p = jnp.log1p(jnp.exp(-ax))
        ls_pos = jnp.minimum(x, 0.0) - l1p
        ls_neg = jnp.minimum(-x, 0.0) - l1p
        ce_f = -(t * ls_pos + (1.0 - t) * ls_neg)
        p = 1.0 / (1.0 + jnp.exp(-x))
        p_t = p * t + (1.0 - p) * (1.0 - t)
        q = 1.0 - p_t
        alpha_t = 0.25 * t + 0.75 * (1.0 - t)
        fl = alpha_t * ce_f * q * q
        acc_ref[2] = jnp.sum(fl * m)
        posm = t * m
        acc_ref[4] = jnp.sum(posm)

        lp = pts_ref[24:32].astype(jnp.float32)
        rp = pts_ref[32:40].astype(jnp.float32)
        lg = pts_ref[40:48].astype(jnp.float32)
        rg = pts_ref[48:56].astype(jnp.float32)
        intsctk = jnp.minimum(rp, rg) + jnp.minimum(lp, lg)
        unionk = (lp + rp) + (lg + rg) - intsctk
        iouk = intsctk / jnp.maximum(unionk, 1e-8)
        len_c = jnp.maximum(lp, lg) + jnp.maximum(rp, rg)
        rho = 0.5 * (rp - lp - rg + lg)
        rr_ = rho / jnp.maximum(len_c, 1e-8)
        dl = 1.0 - iouk + rr_ * rr_
        acc_ref[3] = jnp.sum(dl * posm)

        acc_ref[0] = 0.0
        acc_ref[1] = 0.0

    rl = rr3_ref[pl.ds(j, 1)]
    rr = rr3_ref[pl.ds(j + 8, 1)]
    sc = rr3_ref[pl.ds(j + 16, 1)]
    segc = segc_ref[0]
    gl = segc[:, 0:1]
    gr = segc[:, 1:2]
    min_l = jnp.minimum(gl, rl)
    max_l = jnp.maximum(gl, rl)
    min_r = jnp.minimum(gr, rr)
    max_r = jnp.maximum(gr, rr)
    mat = (min_r - max_l) / (max_r - min_l)
    ious = jnp.max(mat, axis=0, keepdims=True)
    ismax = mat >= ious
    ismax_bf = ismax.astype(jnp.bfloat16)
    cnt = jnp.dot(tri32_ref[...], ismax_bf,
                  preferred_element_type=jnp.float32)
    posf = (ious > _FG_IOU).astype(jnp.float32)
    npos = jnp.sum(posf)
    G = jnp.where(ismax & (cnt < 1.5), posf, 0.0)
    bgf = jnp.where((ious < _BG_IOU) & (sc > 0.0), 1.0, 0.0)
    cum = bgf
    for k in (1, 2, 4, 8, 16, 32, 64, 128, 256, 512):
        cum = cum + jnp.concatenate(
            [jnp.zeros((1, k), jnp.float32), cum[:, :-k]], axis=1)
    bg_sel = bgf * (cum < npos + 0.5).astype(jnp.float32)
    sel = jnp.maximum(posf, bg_sel)

    ce_acc = jnp.float32(0.0)
    for q in range(4):
        n = _QLEN[q]
        mq = rmax_ref[0][:, q * _RPAD:q * _RPAD + n]
        eq = esum_ref[0][:, q * _RPAD:q * _RPAD + n]
        cq = col0_ref[0][:, q * _RPAD:q * _RPAD + n]
        lseq = mq + jnp.log(eq)
        sel_q = sel[:, q * 248:q * 248 + n]
        bg_q = bg_sel[:, q * 248:q * 248 + n]
        G_q = G[:, q * 248:q * 248 + n]
        pq = pick_ref[0, q][:, 0:n]
        ce_acc = ce_acc + (jnp.sum(sel_q * lseq) - jnp.sum(bg_q * cq)
                           - jnp.sum(G_q * pq))
    acc_ref[0] = acc_ref[0] + ce_acc
    acc_ref[1] = acc_ref[1] + jnp.sum(sel)

    @pl.when(j == _B - 1)
    def _fin():
        norm = 90.0 + 0.1 * jnp.maximum(acc_ref[4], 1.0)
        out_ref[0, 0] = (acc_ref[2] + acc_ref[3]) / norm + acc_ref[0] / acc_ref[1]


_TRI32 = np.tri(_Ng, dtype=np.float32).astype(jnp.bfloat16)


def kernel(fpn_masks, out_cls_logits, out_offsets, out_rois, out_scores,
           out_roimask, cls_log, gt_cls, gt_offsets, gt_segments,
           segments_label, segments_mask):
    f32 = jnp.float32
    tri32 = jnp.asarray(_TRI32)

    rmax8, esum8, col08, pick4 = _sc_stats(
        cls_log, segments_label.astype(jnp.int32))

    rr3 = jnp.concatenate(
        [out_rois[:, :, 1], out_rois[:, :, 2], out_scores], axis=0)
    segc = jnp.stack(
        [gt_segments[:, :, 0], gt_segments[:, :, 1],
         segments_label.astype(f32)], axis=2)
    pts = jnp.concatenate(
        [out_cls_logits, gt_cls.astype(f32), fpn_masks.astype(f32),
         out_offsets[:, :, 0], out_offsets[:, :, 1],
         gt_offsets[:, :, 0], gt_offsets[:, :, 1]],
        axis=0).astype(jnp.bfloat16)

    out = pl.pallas_call(
        _body,
        grid=(_B,),
        in_specs=[
            pl.BlockSpec((3 * _B, _Nr), lambda j: (0, 0)),
            pl.BlockSpec((1, _Ng, 3), lambda j: (j, 0, 0)),
            pl.BlockSpec((_Ng, _Ng), lambda j: (0, 0)),
            pl.BlockSpec((1, 1, 4 * _RPAD), lambda j: (j, 0, 0)),
            pl.BlockSpec((1, 1, 4 * _RPAD), lambda j: (j, 0, 0)),
            pl.BlockSpec((1, 1, 4 * _RPAD), lambda j: (j, 0, 0)),
            pl.BlockSpec((1, 4, _Ng, _RPAD), lambda j: (j, 0, 0, 0)),
            pl.BlockSpec((7 * _B, _T), lambda j: (0, 0)),
        ],
        out_specs=pl.BlockSpec((1, 1), lambda j: (0, 0), memory_space=pltpu.SMEM),
        out_shape=jax.ShapeDtypeStruct((1, 1), f32),
        scratch_shapes=[pltpu.SMEM((8,), f32)],
    )(rr3, segc, tri32, rmax8, esum8, col08, pick4, pts)
    return out[0, 0]
